# Initial kernel scaffold; baseline (speedup 1.0000x reference)
import jax
import jax.numpy as jnp
from jax.experimental import pallas as pl


def kernel(U):
    raise NotImplementedError("write your pallas kernel here")


# speedup vs baseline: 1.4352x; 1.4352x over previous
from functools import partial

import jax
import jax.numpy as jnp
from jax.experimental import pallas as pl
from jax.experimental.pallas import tpu as pltpu


def _round_up(x, m):
    return ((x + m - 1) // m) * m


def _repel_kernel(u_ref, out_ref, *, d):
    g_per = 128 // d
    u = u_ref[...]
    inv_sq = pl.reciprocal(u * u, approx=True)
    j_idx = jax.lax.broadcasted_iota(jnp.int32, (128, 128), 0)
    l_idx = jax.lax.broadcasted_iota(jnp.int32, (128, 128), 1)
    m_const = jnp.where(j_idx // d == l_idx % g_per,
                        jnp.float32(71.0), jnp.float32(0.0))
    h = jax.lax.dot_general(
        inv_sq, m_const,
        dimension_numbers=(((1,), (0,)), ((), ())),
        preferred_element_type=jnp.float32,
    )
    rows_per_out = 128 // g_per
    h3 = h.reshape(h.shape[0] // rows_per_out, rows_per_out, 128)
    k_idx = jax.lax.broadcasted_iota(jnp.int32, (rows_per_out, 128), 0)
    l2_idx = jax.lax.broadcasted_iota(jnp.int32, (rows_per_out, 128), 1)
    mask = jnp.where(k_idx == l2_idx // g_per,
                     jnp.float32(1.0), jnp.float32(0.0))
    out_ref[...] = jnp.sum(h3 * mask[None], axis=1)


def kernel(U):
    n, d = U.shape
    orig_dtype = U.dtype
    U32 = U.astype(jnp.float32)

    g_per = 128 // d
    tile_rows = 8192
    packed_n = pl.cdiv(n, g_per)
    num_tiles = pl.cdiv(packed_n, tile_rows)
    padded_packed_n = num_tiles * tile_rows
    padded_n = padded_packed_n * g_per

    flat = U32.reshape(-1)
    pad = padded_n * d - n * d
    if pad:
        flat = jnp.concatenate([flat, jnp.ones((pad,), jnp.float32)])
    packed = flat.reshape(padded_packed_n, 128)

    out_rows_per_tile = tile_rows * g_per // 128
    out = pl.pallas_call(
        partial(_repel_kernel, d=d),
        out_shape=jax.ShapeDtypeStruct((num_tiles * out_rows_per_tile, 128),
                                       jnp.float32),
        grid=(num_tiles,),
        in_specs=[pl.BlockSpec((tile_rows, 128), lambda i: (i, 0))],
        out_specs=pl.BlockSpec((out_rows_per_tile, 128), lambda i: (i, 0)),
        compiler_params=pltpu.CompilerParams(
            dimension_semantics=("parallel",),
            vmem_limit_bytes=64 * 1024 * 1024,
        ),
    )(packed)

    res = out.reshape(padded_n)[:n]
    return res.reshape(n, 1).astype(orig_dtype)


# speedup vs baseline: 8.3406x; 5.8116x over previous
import jax
import jax.numpy as jnp
from jax.experimental import pallas as pl
from jax.experimental.pallas import tpu as pltpu


def _repel_kernel(ut_ref, out_ref):
    ut = ut_ref[...]
    inv_sq = pl.reciprocal(ut * ut, approx=True)
    s = jnp.sum(inv_sq, axis=0, keepdims=True)
    rows = out_ref.shape[0]
    stacked = jnp.concatenate(
        [s[:, k * 128:(k + 1) * 128] for k in range(rows)], axis=0)
    out_ref[...] = stacked * jnp.float32(71.0)


def kernel(U):
    n, d = U.shape
    orig_dtype = U.dtype

    block_n = 8192
    num_blocks = pl.cdiv(n, block_n)
    padded_n = num_blocks * block_n

    ut = U.astype(jnp.float32).T
    if padded_n != n:
        ut = jnp.concatenate(
            [ut, jnp.ones((d, padded_n - n), jnp.float32)], axis=1)

    rows_per_block = block_n // 128
    out = pl.pallas_call(
        _repel_kernel,
        out_shape=jax.ShapeDtypeStruct((padded_n // 128, 128), jnp.float32),
        grid=(num_blocks,),
        in_specs=[pl.BlockSpec((d, block_n), lambda i: (0, i))],
        out_specs=pl.BlockSpec((rows_per_block, 128), lambda i: (i, 0)),
        compiler_params=pltpu.CompilerParams(
            dimension_semantics=("parallel",),
            vmem_limit_bytes=32 * 1024 * 1024,
        ),
    )(ut)

    return out.reshape(padded_n, 1)[:n].astype(orig_dtype)


# speedup vs baseline: 11.8102x; 1.4160x over previous
import jax
import jax.numpy as jnp
from jax.experimental import pallas as pl
from jax.experimental.pallas import tpu as pltpu


def _repel_kernel(ut_ref, out_ref):
    ut = ut_ref[...]
    inv_sq = pl.reciprocal(ut * ut, approx=True)
    s = jnp.sum(inv_sq, axis=0, keepdims=True)
    rows = out_ref.shape[0]
    stacked = jnp.concatenate(
        [s[:, k * 128:(k + 1) * 128] for k in range(rows)], axis=0)
    out_ref[...] = stacked * jnp.float32(71.0)


def kernel(U):
    n, d = U.shape
    orig_dtype = U.dtype

    block_n = 16384
    num_blocks = pl.cdiv(n, block_n)
    padded_n = num_blocks * block_n

    ut = U.astype(jnp.float32).T
    if padded_n != n:
        ut = jnp.concatenate(
            [ut, jnp.ones((d, padded_n - n), jnp.float32)], axis=1)

    rows_per_block = block_n // 128
    out = pl.pallas_call(
        _repel_kernel,
        out_shape=jax.ShapeDtypeStruct((padded_n // 128, 128), jnp.float32),
        grid=(num_blocks,),
        in_specs=[pl.BlockSpec((d, block_n), lambda i: (0, i))],
        out_specs=pl.BlockSpec((rows_per_block, 128), lambda i: (i, 0)),
        compiler_params=pltpu.CompilerParams(
            dimension_semantics=("parallel",),
            vmem_limit_bytes=32 * 1024 * 1024,
        ),
    )(ut)

    return out.reshape(padded_n, 1)[:n].astype(orig_dtype)


# speedup vs baseline: 14.7559x; 1.2494x over previous
import jax
import jax.numpy as jnp
from jax.experimental import pallas as pl
from jax.experimental.pallas import tpu as pltpu


def _repel_kernel(ut_ref, out_ref):
    ut = ut_ref[...]
    inv_sq = pl.reciprocal(ut * ut, approx=True)
    s = jnp.sum(inv_sq, axis=0, keepdims=True)
    rows = out_ref.shape[0]
    stacked = jnp.concatenate(
        [s[:, k * 128:(k + 1) * 128] for k in range(rows)], axis=0)
    out_ref[...] = stacked * jnp.float32(71.0)


def kernel(U):
    n, d = U.shape
    orig_dtype = U.dtype

    block_n = 32768
    num_blocks = pl.cdiv(n, block_n)
    padded_n = num_blocks * block_n

    ut = U.astype(jnp.float32).T
    if padded_n != n:
        ut = jnp.concatenate(
            [ut, jnp.ones((d, padded_n - n), jnp.float32)], axis=1)

    rows_per_block = block_n // 128
    out = pl.pallas_call(
        _repel_kernel,
        out_shape=jax.ShapeDtypeStruct((padded_n // 128, 128), jnp.float32),
        grid=(num_blocks,),
        in_specs=[pl.BlockSpec((d, block_n), lambda i: (0, i))],
        out_specs=pl.BlockSpec((rows_per_block, 128), lambda i: (i, 0)),
        compiler_params=pltpu.CompilerParams(
            dimension_semantics=("parallel",),
            vmem_limit_bytes=32 * 1024 * 1024,
        ),
    )(ut)

    return out.reshape(padded_n, 1)[:n].astype(orig_dtype)


# speedup vs baseline: 15.7980x; 1.0706x over previous
import jax
import jax.numpy as jnp
from jax.experimental import pallas as pl
from jax.experimental.pallas import tpu as pltpu


def _repel_kernel(ut_ref, out_ref):
    ut = ut_ref[...]
    inv_sq = pl.reciprocal(ut * ut, approx=True)
    s = jnp.sum(inv_sq, axis=0, keepdims=True)
    rows = out_ref.shape[0]
    stacked = jnp.concatenate(
        [s[:, k * 128:(k + 1) * 128] for k in range(rows)], axis=0)
    out_ref[...] = stacked * jnp.float32(71.0)


def kernel(U):
    n, d = U.shape
    orig_dtype = U.dtype

    block_n = 65536
    num_blocks = pl.cdiv(n, block_n)
    padded_n = num_blocks * block_n

    ut = U.astype(jnp.float32).T
    if padded_n != n:
        ut = jnp.concatenate(
            [ut, jnp.ones((d, padded_n - n), jnp.float32)], axis=1)

    rows_per_block = block_n // 128
    out = pl.pallas_call(
        _repel_kernel,
        out_shape=jax.ShapeDtypeStruct((padded_n // 128, 128), jnp.float32),
        grid=(num_blocks,),
        in_specs=[pl.BlockSpec((d, block_n), lambda i: (0, i))],
        out_specs=pl.BlockSpec((rows_per_block, 128), lambda i: (i, 0)),
        compiler_params=pltpu.CompilerParams(
            dimension_semantics=("parallel",),
            vmem_limit_bytes=32 * 1024 * 1024,
        ),
    )(ut)

    return out.reshape(padded_n, 1)[:n].astype(orig_dtype)
